# baseline (speedup 1.0000x reference)
import jax
import jax.numpy as jnp
from jax.experimental import pallas as pl
from jax.experimental.pallas import tpu as pltpu

_RADIUS = 2
_WIN = 2 * _RADIUS + 1
_PAT = 6016
_TILE_R = 112


def _body(idx_ref, fields_ref, onehot_ref, out_ref, win_ref, pat_ref, sem):
    b = pl.program_id(0)
    t = pl.program_id(1)

    @pl.when(t == 0)
    def _():
        py = idx_ref[b, 0]
        px = idx_ref[b, 1]
        cp = pltpu.make_async_copy(
            fields_ref.at[b, pl.ds(py - _RADIUS, _WIN), pl.ds(px - _RADIUS, _WIN), :],
            win_ref,
            sem,
        )
        cp.start()
        cp.wait()
        w = win_ref[:, :, 2:]
        mean2d = jnp.sum(w, axis=(0, 1)).reshape(1, -1) * (1.0 / (_WIN * _WIN))
        pat_ref[...] = jnp.dot(
            mean2d, onehot_ref[...], preferred_element_type=jnp.float32
        )

    out_ref[...] = jnp.broadcast_to(pat_ref[...][:, None, :], out_ref.shape)


def kernel(fields, pump_indices):
    B, H, W, C = fields.shape
    Cout = C - 2
    idx = pump_indices.astype(jnp.int32)
    n_rows = H * W * Cout // _PAT

    onehot = (
        jnp.arange(_PAT, dtype=jnp.int32)[None, :] % Cout
        == jnp.arange(Cout, dtype=jnp.int32)[:, None]
    ).astype(jnp.float32)

    grid_spec = pltpu.PrefetchScalarGridSpec(
        num_scalar_prefetch=1,
        grid=(B, n_rows // _TILE_R),
        in_specs=[
            pl.BlockSpec(memory_space=pl.ANY),
            pl.BlockSpec((Cout, _PAT), lambda b, t, idx_ref: (0, 0)),
        ],
        out_specs=pl.BlockSpec(
            (1, _TILE_R, _PAT), lambda b, t, idx_ref: (b, t, 0)
        ),
        scratch_shapes=[
            pltpu.VMEM((_WIN, _WIN, C), jnp.float32),
            pltpu.VMEM((1, _PAT), jnp.float32),
            pltpu.SemaphoreType.DMA,
        ],
    )

    flat = pl.pallas_call(
        _body,
        grid_spec=grid_spec,
        out_shape=jax.ShapeDtypeStruct((B, n_rows, _PAT), jnp.float32),
    )(idx, fields, onehot)
    return flat.reshape(B, H, W, Cout)


# speedup vs baseline: 1.6926x; 1.6926x over previous
import jax
import jax.numpy as jnp
from jax.experimental import pallas as pl
from jax.experimental.pallas import tpu as pltpu

_RADIUS = 2
_WIN = 2 * _RADIUS + 1
_CHUNK_H = 28
_NCHUNK = 8


def _make_body(B, H, W, C):
    Cout = C - 2

    def _body(idx_ref, fields_ref, out_ref, winbuf, buf, wsem, osem):
        b = pl.program_id(0)
        nb = pl.num_programs(0)
        par = jax.lax.rem(b, 2)

        def window_copy(bb):
            py = idx_ref[bb, 0]
            px = idx_ref[bb, 1]
            return pltpu.make_async_copy(
                fields_ref.at[
                    bb, pl.ds(py - _RADIUS, _WIN), pl.ds(px - _RADIUS, _WIN), :
                ],
                winbuf.at[bb],
                wsem,
            )

        def chunk_copy(bb, pp, c):
            return pltpu.make_async_copy(
                buf.at[pp],
                out_ref.at[bb, pl.ds(c * _CHUNK_H, _CHUNK_H), :, :],
                osem.at[pp, c],
            )

        @pl.when(b == 0)
        def _():
            for bb in range(B):
                window_copy(bb).start()
            for bb in range(B):
                window_copy(bb).wait()

        @pl.when(b >= 2)
        def _():
            for c in range(_NCHUNK):
                chunk_copy(b - 2, par, c).wait()

        w = winbuf[b, :, :, 2:]
        mean = jnp.sum(w, axis=(0, 1)) * (1.0 / (_WIN * _WIN))
        buf[par] = jnp.broadcast_to(mean[None, None, :], (_CHUNK_H, W, Cout))

        for c in range(_NCHUNK):
            chunk_copy(b, par, c).start()

        @pl.when(b == nb - 1)
        def _():
            for c in range(_NCHUNK):
                chunk_copy(b - 1, 1 - par, c).wait()
            for c in range(_NCHUNK):
                chunk_copy(b, par, c).wait()

    return _body


def kernel(fields, pump_indices):
    B, H, W, C = fields.shape
    Cout = C - 2
    idx = pump_indices.astype(jnp.int32)

    grid_spec = pltpu.PrefetchScalarGridSpec(
        num_scalar_prefetch=1,
        grid=(B,),
        in_specs=[
            pl.BlockSpec(memory_space=pl.ANY),
        ],
        out_specs=pl.BlockSpec(memory_space=pl.ANY),
        scratch_shapes=[
            pltpu.VMEM((B, _WIN, _WIN, C), jnp.float32),
            pltpu.VMEM((2, _CHUNK_H, W, Cout), jnp.float32),
            pltpu.SemaphoreType.DMA,
            pltpu.SemaphoreType.DMA((2, _NCHUNK)),
        ],
    )

    return pl.pallas_call(
        _make_body(B, H, W, C),
        grid_spec=grid_spec,
        out_shape=jax.ShapeDtypeStruct((B, H, W, Cout), jnp.float32),
    )(idx, fields)
